# baseline (speedup 1.0000x reference)
import math
import functools

import jax
import jax.numpy as jnp
from jax.experimental import pallas as pl
from jax.experimental.pallas import tpu as pltpu

IMG = 512
SIZES = (32.0, 64.0, 128.0, 256.0, 512.0)
RATIOS = (0.5, 1.0, 2.0)
PRE_NMS_TOP_N = 1000
POST_NMS_TOP_N = 1000
NMS_THRESH = 0.7
MIN_SIZE = 1e-2
BBOX_XFORM_CLIP = math.log(1000.0 / 16.0)


def _rpn_head_row_kernel(r0, r1, r2, wtap, headw, headb, convb, out):
    W = out.shape[1]
    rows = (r0[0, :, :], r1[0, :, :], r2[0, :, :])
    acc = jnp.zeros((W, 256), dtype=jnp.float32)
    for dy in range(3):
        for dx in range(3):
            acc = acc + jnp.dot(rows[dy][dx:dx + W, :], wtap[dy * 3 + dx, :, :],
                                preferred_element_type=jnp.float32)
    t = jnp.maximum(acc + convb[0, :], 0.0)
    out[0, :, :] = jnp.dot(t, headw[:, :], preferred_element_type=jnp.float32) + headb[0, :]


def _rpn_head_level(xpad, wtap, headw, headb, convb):
    Hp, Wp, _ = xpad.shape
    H, W = Hp - 2, Wp - 2
    row_spec = lambda off: pl.BlockSpec((1, Wp, 256), lambda i: (i + off, 0, 0))
    return pl.pallas_call(
        _rpn_head_row_kernel,
        grid=(H,),
        in_specs=[
            row_spec(0), row_spec(1), row_spec(2),
            pl.BlockSpec((9, 256, 256), lambda i: (0, 0, 0)),
            pl.BlockSpec((256, 128), lambda i: (0, 0)),
            pl.BlockSpec((1, 128), lambda i: (0, 0)),
            pl.BlockSpec((1, 256), lambda i: (0, 0)),
        ],
        out_specs=pl.BlockSpec((1, W, 128), lambda i: (i, 0, 0)),
        out_shape=jax.ShapeDtypeStruct((H, W, 128), jnp.float32),
    )(xpad, xpad, xpad, wtap, headw, headb, convb)


def _grid_anchors(H, W, stride, size):
    r = jnp.array(RATIOS, dtype=jnp.float32)
    h_r = jnp.sqrt(r)
    w_r = 1.0 / h_r
    ws = size * w_r
    hs = size * h_r
    base = jnp.stack([-ws / 2.0, -hs / 2.0, ws / 2.0, hs / 2.0], axis=1)
    sy = jnp.arange(H, dtype=jnp.float32) * stride
    sx = jnp.arange(W, dtype=jnp.float32) * stride
    yy, xx = jnp.meshgrid(sy, sx, indexing='ij')
    shifts = jnp.stack([xx.reshape(-1), yy.reshape(-1), xx.reshape(-1), yy.reshape(-1)], axis=1)
    return (shifts[:, None, :] + base[None, :, :]).reshape(-1, 4)


def _decode_boxes(deltas, anchors):
    w = anchors[..., 2] - anchors[..., 0]
    h = anchors[..., 3] - anchors[..., 1]
    cx = anchors[..., 0] + 0.5 * w
    cy = anchors[..., 1] + 0.5 * h
    dx, dy, dw, dh = deltas[..., 0], deltas[..., 1], deltas[..., 2], deltas[..., 3]
    dw = jnp.minimum(dw, BBOX_XFORM_CLIP)
    dh = jnp.minimum(dh, BBOX_XFORM_CLIP)
    pcx = dx * w + cx
    pcy = dy * h + cy
    pw = jnp.exp(dw) * w
    ph = jnp.exp(dh) * h
    return jnp.stack([pcx - 0.5 * pw, pcy - 0.5 * ph, pcx + 0.5 * pw, pcy + 0.5 * ph], axis=-1)


def _nms_image(boxes, scores, lvls):
    off = lvls.astype(jnp.float32)[:, None] * (IMG * 4.0)
    b = boxes + off
    order = jnp.argsort(-scores)
    bs = jax.lax.stop_gradient(b[order])
    ss = scores[order]
    area = (bs[:, 2] - bs[:, 0]) * (bs[:, 3] - bs[:, 1])
    lt = jnp.maximum(bs[:, None, :2], bs[None, :, :2])
    rb = jnp.minimum(bs[:, None, 2:], bs[None, :, 2:])
    wh = jnp.maximum(rb - lt, 0.0)
    inter = wh[..., 0] * wh[..., 1]
    iou = inter / (area[:, None] + area[None, :] - inter + 1e-9)
    N = boxes.shape[0]
    idxs = jnp.arange(N)

    def body(keep, i):
        sup = (iou[i] > NMS_THRESH) & (idxs > i) & keep[i]
        return keep & (~sup), None

    keep, _ = jax.lax.scan(body, jnp.ones((N,), dtype=bool), idxs)
    fs = jnp.where(keep, ss, -jnp.inf)
    top_s, top_i = jax.lax.top_k(fs, POST_NMS_TOP_N)
    fb = boxes[order[top_i]]
    return fb


def kernel(images, feat0, feat1, feat2, feat3, feat4, conv_w, conv_b, cls_w, cls_b, bbox_w, bbox_b):
    feats = [feat0, feat1, feat2, feat3, feat4]
    B = feats[0].shape[0]

    wtap = jnp.transpose(conv_w, (2, 3, 1, 0)).reshape(9, 256, 256)
    headw = jnp.zeros((256, 128), jnp.float32)
    headw = headw.at[:, 0:3].set(cls_w[:, :, 0, 0].T)
    headw = headw.at[:, 3:15].set(bbox_w[:, :, 0, 0].T)
    headb = jnp.zeros((128,), jnp.float32)
    headb = headb.at[0:3].set(cls_b).at[3:15].set(bbox_b)
    headb = headb.reshape(1, 128)
    convb = conv_b.reshape(1, 256)

    all_scores, all_boxes, all_lvls = [], [], []
    for lvl, f in enumerate(feats):
        H, W = f.shape[2], f.shape[3]
        stride = IMG // H
        x = jnp.transpose(f[0], (1, 2, 0))
        xpad = jnp.pad(x, ((1, 1), (1, 1), (0, 0)))
        head = _rpn_head_level(xpad, wtap, headw, headb, convb)
        obj = head[..., 0:3].reshape(1, -1)
        dl = head[..., 3:15].reshape(1, -1, 4)
        anchors = _grid_anchors(H, W, float(stride), SIZES[lvl])
        props = _decode_boxes(dl, anchors[None, :, :])
        k = min(PRE_NMS_TOP_N, obj.shape[1])
        top_v, top_i = jax.lax.top_k(obj, k)
        pb = jnp.take_along_axis(props, top_i[..., None], axis=1)
        all_scores.append(top_v)
        all_boxes.append(pb)
        all_lvls.append(jnp.full((B, k), lvl, dtype=jnp.int32))
    scores = jnp.concatenate(all_scores, axis=1)
    boxes = jnp.concatenate(all_boxes, axis=1)
    lvls = jnp.concatenate(all_lvls, axis=1)
    boxes = jnp.clip(boxes, 0.0, float(IMG))
    probs = jax.nn.sigmoid(scores)
    ws = boxes[..., 2] - boxes[..., 0]
    hs = boxes[..., 3] - boxes[..., 1]
    valid = (ws >= MIN_SIZE) & (hs >= MIN_SIZE)
    probs = jnp.where(valid, probs, -1e9)
    fb = jax.vmap(_nms_image)(boxes, probs, lvls)
    return fb
